# baseline (speedup 1.0000x reference)
import jax
import jax.numpy as jnp
from jax import lax
from jax.experimental import pallas as pl
from jax.experimental.pallas import tpu as pltpu
from jax.experimental.pallas import tpu_sc as plsc

_N = 50000
_D = 256
_G = 64
_L = 16
_NV = _D // _L
_C = 128
_NEG = float("-inf")


def _lane_extract(vec, lane):
    lid = lax.broadcasted_iota(jnp.int32, (_L,), 0)
    return jnp.max(jnp.where(lid == lane, vec, 0))


def _sc_body(x_hbm, batch_hbm, out_hbm, batch_v, bnd_v, bsh, bndsh,
             buf0, buf1, accv, sem0, sem1):
    sid = lax.axis_index("s")
    wid = sid * 2 + lax.axis_index("c")
    lid = lax.broadcasted_iota(jnp.int32, (_L,), 0)

    @pl.when(sid == 0)
    def _stage():
        pltpu.sync_copy(batch_hbm, bsh)
        pltpu.sync_copy(bsh, batch_v)
        for t in range(5):
            tgt = 16 * t + lid
            lo = jnp.zeros((_L,), jnp.int32)
            hi = jnp.full((_L,), _N, jnp.int32)

            def bs_step(_, carry, tgt=tgt):
                lo, hi = carry
                active = lo < hi
                mid = lax.shift_right_logical(lo + hi, 1)
                v = plsc.load_gather(batch_v, [jnp.minimum(mid, _N - 1)])
                less = v < tgt
                lo = jnp.where(active & less, mid + 1, lo)
                hi = jnp.where(active & (~less), mid, hi)
                return lo, hi

            lo, _ = lax.fori_loop(0, 16, bs_step, (lo, hi))
            bnd_v[pl.ds(16 * t, _L)] = lo
        pltpu.sync_copy(bnd_v, bndsh)

    plsc.subcore_barrier()
    pltpu.sync_copy(bndsh, bnd_v)
    bv = bnd_v[pl.ds(2 * wid, _L)]
    s0 = _lane_extract(bv, 0)
    sm = _lane_extract(bv, 1)
    s1 = _lane_extract(bv, 2)

    a0 = jnp.bitwise_and(s0, -8)
    top = jnp.bitwise_and(s1 + 7, -8)
    nchunks = (top - a0 + _C - 1) // _C

    def off(k):
        o = jnp.maximum(0, jnp.minimum(a0 + k * _C, top - _C))
        return pl.multiple_of(o, 8)

    def start(k, buf, sem):
        pltpu.make_async_copy(x_hbm.at[pl.ds(off(k), _C)], buf, sem).start()

    @pl.when(nchunks > 0)
    def _p0():
        start(0, buf0, sem0)

    @pl.when(nchunks > 1)
    def _p1():
        start(1, buf1, sem1)

    neg = jnp.full((_L,), _NEG, jnp.float32)
    acc = [neg] * (2 * _NV)

    def make_row_body(buf, base):
        def row_body(r, a):
            a = list(a)
            for j in range(_NV):
                a[base + j] = jnp.maximum(a[base + j], buf[r, pl.ds(j * _L, _L)])
            return tuple(a)
        return row_body

    def chunk(k, buf, sem, acc):
        @pl.when(k < nchunks)
        def _w():
            pltpu.make_async_copy(
                x_hbm.at[pl.ds(off(k), _C)], buf, sem).wait()

        valid = k < nchunks
        o = off(k)
        i0 = jnp.where(valid, jnp.clip(s0 - o, 0, _C), 0)
        i1 = jnp.where(valid, jnp.clip(sm - o, 0, _C), 0)
        i2 = jnp.where(valid, jnp.clip(s1 - o, 0, _C), 0)
        acc = lax.fori_loop(i0, i1, make_row_body(buf, 0), acc)
        acc = lax.fori_loop(i1, i2, make_row_body(buf, _NV), acc)

        @pl.when(k + 2 < nchunks)
        def _n():
            start(k + 2, buf, sem)

        return acc

    def pair(p, acc):
        acc = chunk(2 * p, buf0, sem0, acc)
        acc = chunk(2 * p + 1, buf1, sem1, acc)
        return acc

    acc = lax.fori_loop(0, (nchunks + 1) // 2, pair, tuple(acc))

    for j in range(_NV):
        accv[pl.ds(j * _L, _L)] = acc[j]
        accv[pl.ds(_D + j * _L, _L)] = acc[_NV + j]
    pltpu.sync_copy(accv, out_hbm.at[pl.ds(wid * 2 * _D, 2 * _D)])


@jax.jit
def kernel(x, batch):
    mesh = plsc.VectorSubcoreMesh(core_axis_name="c", subcore_axis_name="s")
    f = pl.kernel(
        _sc_body,
        out_type=jax.ShapeDtypeStruct((_G * _D,), jnp.float32),
        mesh=mesh,
        compiler_params=pltpu.CompilerParams(needs_layout_passes=False),
        scratch_types=[
            pltpu.VMEM((_N,), jnp.int32),
            pltpu.VMEM((80,), jnp.int32),
            pltpu.VMEM_SHARED((_N,), jnp.int32),
            pltpu.VMEM_SHARED((80,), jnp.int32),
            pltpu.VMEM((_C, _D), jnp.float32),
            pltpu.VMEM((_C, _D), jnp.float32),
            pltpu.VMEM((2 * _D,), jnp.float32),
            pltpu.SemaphoreType.DMA,
            pltpu.SemaphoreType.DMA,
        ],
    )
    return f(x, batch).reshape(_G, _D)


# speedup vs baseline: 1.0182x; 1.0182x over previous
import jax
import jax.numpy as jnp
from jax import lax
from jax.experimental import pallas as pl
from jax.experimental.pallas import tpu as pltpu
from jax.experimental.pallas import tpu_sc as plsc

_N = 50000
_D = 256
_G = 64
_L = 16
_NV = _D // _L
_C = 144
_NEG = float("-inf")


def _lane_extract(vec, lane):
    lid = lax.broadcasted_iota(jnp.int32, (_L,), 0)
    return jnp.max(jnp.where(lid == lane, vec, 0))


def _sc_body(x_hbm, batch_hbm, out_hbm, batch_v, bsh, buf0, buf1, accv,
             sem0, sem1):
    sid = lax.axis_index("s")
    wid = sid * 2 + lax.axis_index("c")

    @pl.when(sid == 0)
    def _stage():
        pltpu.sync_copy(batch_hbm, bsh)

    plsc.subcore_barrier()
    pltpu.sync_copy(bsh, batch_v)
    lid = lax.broadcasted_iota(jnp.int32, (_L,), 0)
    tgt = 2 * wid + lid
    lo = jnp.zeros((_L,), jnp.int32)
    hi = jnp.full((_L,), _N, jnp.int32)

    def bs_step(_, carry):
        lo, hi = carry
        active = lo < hi
        mid = lax.shift_right_logical(lo + hi, 1)
        v = plsc.load_gather(batch_v, [jnp.minimum(mid, _N - 1)])
        less = v < tgt
        lo = jnp.where(active & less, mid + 1, lo)
        hi = jnp.where(active & (~less), mid, hi)
        return lo, hi

    lo, _ = lax.fori_loop(0, 16, bs_step, (lo, hi))
    s0 = _lane_extract(lo, 0)
    sm = _lane_extract(lo, 1)
    s1 = _lane_extract(lo, 2)

    a0 = jnp.bitwise_and(s0, -8)
    top = jnp.bitwise_and(s1 + 7, -8)
    nchunks = (top - a0 + _C - 1) // _C

    def off(k):
        o = jnp.maximum(0, jnp.minimum(a0 + k * _C, top - _C))
        return pl.multiple_of(o, 8)

    def start(k, buf, sem):
        pltpu.make_async_copy(x_hbm.at[pl.ds(off(k), _C)], buf, sem).start()

    @pl.when(nchunks > 0)
    def _p0():
        start(0, buf0, sem0)

    @pl.when(nchunks > 1)
    def _p1():
        start(1, buf1, sem1)

    neg = jnp.full((_L,), _NEG, jnp.float32)
    acc = [neg] * (2 * _NV)

    def make_row_body(buf, base):
        def row_body(r, a):
            a = list(a)
            for j in range(_NV):
                a[base + j] = jnp.maximum(a[base + j], buf[r, pl.ds(j * _L, _L)])
            return tuple(a)
        return row_body

    def chunk(k, buf, sem, acc):
        @pl.when(k < nchunks)
        def _w():
            pltpu.make_async_copy(
                x_hbm.at[pl.ds(off(k), _C)], buf, sem).wait()

        valid = k < nchunks
        o = off(k)
        i0 = jnp.where(valid, jnp.clip(s0 - o, 0, _C), 0)
        i1 = jnp.where(valid, jnp.clip(sm - o, 0, _C), 0)
        i2 = jnp.where(valid, jnp.clip(s1 - o, 0, _C), 0)
        acc = lax.fori_loop(i0, i1, make_row_body(buf, 0), acc)
        acc = lax.fori_loop(i1, i2, make_row_body(buf, _NV), acc)

        @pl.when(k + 2 < nchunks)
        def _n():
            start(k + 2, buf, sem)

        return acc

    def pair(p, acc):
        acc = chunk(2 * p, buf0, sem0, acc)
        acc = chunk(2 * p + 1, buf1, sem1, acc)
        return acc

    acc = lax.fori_loop(0, (nchunks + 1) // 2, pair, tuple(acc))

    for j in range(_NV):
        accv[pl.ds(j * _L, _L)] = acc[j]
        accv[pl.ds(_D + j * _L, _L)] = acc[_NV + j]
    pltpu.sync_copy(accv, out_hbm.at[pl.ds(wid * 2 * _D, 2 * _D)])


@jax.jit
def kernel(x, batch):
    mesh = plsc.VectorSubcoreMesh(core_axis_name="c", subcore_axis_name="s")
    f = pl.kernel(
        _sc_body,
        out_type=jax.ShapeDtypeStruct((_G * _D,), jnp.float32),
        mesh=mesh,
        compiler_params=pltpu.CompilerParams(needs_layout_passes=False),
        scratch_types=[
            pltpu.VMEM((_N,), jnp.int32),
            pltpu.VMEM_SHARED((_N,), jnp.int32),
            pltpu.VMEM((_C, _D), jnp.float32),
            pltpu.VMEM((_C, _D), jnp.float32),
            pltpu.VMEM((2 * _D,), jnp.float32),
            pltpu.SemaphoreType.DMA,
            pltpu.SemaphoreType.DMA,
        ],
    )
    return f(x, batch).reshape(_G, _D)
